# baseline (speedup 1.0000x reference)
import functools

import jax
import jax.numpy as jnp
from jax import lax
from jax.experimental import pallas as pl
from jax.experimental.pallas import tpu as pltpu
import jax.experimental.pallas.tpu_sc as plsc

N_NODES = 100000
MEM_DIM = 128
MSG_DIM = 256
B = 16384

NC = 2
NS = 16
NW = NC * NS
BPW = B // NW
CHUNK = 128
NCHUNK = BPW // CHUNK

CPW = 3128
CT = 136
CK = 22
CT_LAST = 40

LUW = 3200
LU_PAD = NW * LUW

_sc_mesh = plsc.VectorSubcoreMesh(core_axis_name="c", subcore_axis_name="s")
_sc_params = pltpu.CompilerParams(needs_layout_passes=False)


def _wid():
  return lax.axis_index("s") * NC + lax.axis_index("c")


@functools.partial(
    pl.kernel,
    mesh=_sc_mesh,
    out_type=(
        jax.ShapeDtypeStruct((B, MEM_DIM), jnp.float32),
        jax.ShapeDtypeStruct((N_NODES, MEM_DIM), jnp.float32),
    ),
    scratch_types=[
        pltpu.VMEM((NCHUNK, CHUNK), jnp.int32),
        pltpu.VMEM((BPW, MEM_DIM), jnp.float32),
        pltpu.VMEM((CT, MEM_DIM), jnp.float32),
        pltpu.VMEM((CT, MEM_DIM), jnp.float32),
        pltpu.SemaphoreType.DMA,
        pltpu.SemaphoreType.DMA,
        pltpu.SemaphoreType.DMA,
        pltpu.SemaphoreType.DMA,
        pltpu.SemaphoreType.DMA,
    ],
)
def _sc_gather_clone(table, idx3, h_out, mem_out, idx_v, rows_v,
                     c0, c1, sem_g, sr0, sr1, sw0, sw1):
  wid = _wid()
  base = wid * BPW
  crow = wid * CPW
  pltpu.sync_copy(idx3.at[wid], idx_v)
  gathers = [
      pltpu.async_copy(table.at[idx_v.at[c]],
                       rows_v.at[pl.ds(c * CHUNK, CHUNK)], sem_g)
      for c in range(NCHUNK)
  ]

  bufs = (c0, c1)
  srs = (sr0, sr1)
  sws = (sw0, sw1)

  def rd(i):
    return pltpu.async_copy(table.at[pl.ds(crow + i * CT, CT)],
                            bufs[i % 2], srs[i % 2])

  def wr(i):
    return pltpu.async_copy(bufs[i % 2],
                            mem_out.at[pl.ds(crow + i * CT, CT)], sws[i % 2])

  reads = [None] * CK
  writes = [None] * CK
  reads[0] = rd(0)
  reads[1] = rd(1)
  for i in range(CK):
    reads[i].wait()
    writes[i] = wr(i)
    if i + 2 < CK:
      writes[i].wait()
      reads[i + 2] = rd(i + 2)
  writes[CK - 2].wait()
  writes[CK - 1].wait()

  tail = crow + CK * CT

  @pl.when(wid < NW - 1)
  def _():
    pltpu.sync_copy(table.at[pl.ds(tail, CT)], c0)
    pltpu.sync_copy(c0, mem_out.at[pl.ds(tail, CT)])

  @pl.when(wid == NW - 1)
  def _():
    pltpu.sync_copy(table.at[pl.ds(tail, CT_LAST)], c0.at[pl.ds(0, CT_LAST)])
    pltpu.sync_copy(c0.at[pl.ds(0, CT_LAST)], mem_out.at[pl.ds(tail, CT_LAST)])

  for cp in gathers:
    cp.wait()
  pltpu.sync_copy(rows_v, h_out.at[pl.ds(base, BPW)])


_BLK = 1024


def _gru_body(msg_ref, h_ref, wih_ref, whh_ref, bih_ref, bhh_ref, out_ref):
  x = msg_ref[...]
  h = h_ref[...]
  dn = (((1,), (1,)), ((), ()))
  gi = lax.dot_general(x, wih_ref[...], dn,
                       preferred_element_type=jnp.float32) + bih_ref[...]
  gh = lax.dot_general(h, whh_ref[...], dn,
                       preferred_element_type=jnp.float32) + bhh_ref[...]
  i_r, i_z, i_n = gi[:, :128], gi[:, 128:256], gi[:, 256:]
  h_r, h_z, h_n = gh[:, :128], gh[:, 128:256], gh[:, 256:]
  r = jax.nn.sigmoid(i_r + h_r)
  z = jax.nn.sigmoid(i_z + h_z)
  n = jnp.tanh(i_n + r * h_n)
  out_ref[...] = (1.0 - z) * n + z * h


def _tc_gru(msg, h, w_ih, w_hh, b_ih, b_hh):
  return pl.pallas_call(
      _gru_body,
      grid=(B // _BLK,),
      in_specs=[
          pl.BlockSpec((_BLK, MSG_DIM), lambda i: (i, 0)),
          pl.BlockSpec((_BLK, MEM_DIM), lambda i: (i, 0)),
          pl.BlockSpec((3 * MEM_DIM, MSG_DIM), lambda i: (0, 0)),
          pl.BlockSpec((3 * MEM_DIM, MEM_DIM), lambda i: (0, 0)),
          pl.BlockSpec((3 * MEM_DIM,), lambda i: (0,)),
          pl.BlockSpec((3 * MEM_DIM,), lambda i: (0,)),
      ],
      out_specs=pl.BlockSpec((_BLK, MEM_DIM), lambda i: (i, 0)),
      out_shape=jax.ShapeDtypeStruct((B, MEM_DIM), jnp.float32),
  )(msg, h, w_ih, w_hh, b_ih, b_hh)


@functools.partial(
    pl.kernel,
    mesh=_sc_mesh,
    out_type=jax.ShapeDtypeStruct((LU_PAD,), jnp.float32),
    scratch_types=[
        pltpu.VMEM((NCHUNK, CHUNK), jnp.int32),
        pltpu.VMEM((BPW, MEM_DIM), jnp.float32),
        pltpu.VMEM((NW, NCHUNK, CHUNK), jnp.int32),
        pltpu.VMEM((LUW,), jnp.float32),
        pltpu.VMEM((16,), jnp.float32),
        pltpu.SemaphoreType.DMA,
    ],
    compiler_params=_sc_params,
)
def _sc_scatter_lu(idx3, rows, lu_pad, tvec_hbm, mem_ref, lu_out,
                   idxc_v, rows_v, idx_v, seg_v, tv_v, sem):
  wid = _wid()
  base = wid * BPW
  lo = wid * LUW
  pltpu.sync_copy(idx3.at[wid], idxc_v)
  pltpu.sync_copy(rows.at[pl.ds(base, BPW)], rows_v)
  scatters = [
      pltpu.async_copy(rows_v.at[pl.ds(c * CHUNK, CHUNK)],
                       mem_ref.at[idxc_v.at[c]], sem)
      for c in range(NCHUNK)
  ]

  pltpu.sync_copy(idx3, idx_v)
  pltpu.sync_copy(lu_pad.at[pl.ds(lo, LUW)], seg_v)
  pltpu.sync_copy(tvec_hbm, tv_v)
  tv = tv_v[...]
  lov = jnp.full((16,), lo, jnp.int32)
  hiv = lov + LUW

  @pl.loop(0, B // 16)
  def _(i):
    w = i // (NCHUNK * CHUNK // 16)
    rem = i % (NCHUNK * CHUNK // 16)
    c = rem // (CHUNK // 16)
    j = rem % (CHUNK // 16)
    iv = idx_v[w, c, pl.ds(j * 16, 16)]
    m = (iv >= lov) & (iv < hiv)
    plsc.store_scatter(seg_v, [iv - lov], tv, mask=m)

  pltpu.sync_copy(seg_v, lu_out.at[pl.ds(lo, LUW)])

  for cp in scatters:
    cp.wait()


def kernel(unique_nids, unique_msg, time, memory, last_update,
           W_ih, W_hh, b_ih, b_hh):
  idx = unique_nids.astype(jnp.int32)
  idx3 = idx.reshape(NW, NCHUNK, CHUNK)
  tvec = jnp.full((16,), time, dtype=jnp.float32)
  lu_pad = jnp.zeros((LU_PAD,), jnp.float32).at[:N_NODES].set(last_update)

  h, mem_copy = _sc_gather_clone(memory, idx3)
  h_new = _tc_gru(unique_msg, h, W_ih, W_hh, b_ih, b_hh)

  mem_ref = jax.new_ref(mem_copy)
  lu_out = _sc_scatter_lu(idx3, h_new, lu_pad, tvec, mem_ref)
  updated_memory = jax.freeze(mem_ref)
  updated_last_update = lu_out[:N_NODES]
  return (updated_memory, updated_last_update)


# speedup vs baseline: 1.0622x; 1.0622x over previous
import functools

import jax
import jax.numpy as jnp
from jax import lax
from jax.experimental import pallas as pl
from jax.experimental.pallas import tpu as pltpu
import jax.experimental.pallas.tpu_sc as plsc

N_NODES = 100000
MEM_DIM = 128
MSG_DIM = 256
B = 16384

NC = 2
NS = 16
NW = NC * NS
BPW = B // NW
CHUNK = 128
NCHUNK = BPW // CHUNK

CPW = 3128
CT = 136
CK = 22
CT_LAST = 40

LUW = 3200
LUW_LAST = N_NODES - (NW - 1) * LUW

_sc_mesh = plsc.VectorSubcoreMesh(core_axis_name="c", subcore_axis_name="s")
_sc_params = pltpu.CompilerParams(needs_layout_passes=False)


def _wid():
  return lax.axis_index("s") * NC + lax.axis_index("c")


@functools.partial(
    pl.kernel,
    mesh=_sc_mesh,
    out_type=(
        jax.ShapeDtypeStruct((B, MEM_DIM), jnp.float32),
        jax.ShapeDtypeStruct((N_NODES, MEM_DIM), jnp.float32),
    ),
    scratch_types=[
        pltpu.VMEM((NCHUNK, CHUNK), jnp.int32),
        pltpu.VMEM((BPW, MEM_DIM), jnp.float32),
        pltpu.VMEM((CT, MEM_DIM), jnp.float32),
        pltpu.VMEM((CT, MEM_DIM), jnp.float32),
        pltpu.SemaphoreType.DMA,
        pltpu.SemaphoreType.DMA,
        pltpu.SemaphoreType.DMA,
        pltpu.SemaphoreType.DMA,
        pltpu.SemaphoreType.DMA,
    ],
)
def _sc_gather_clone(table, idx3, h_out, mem_out, idx_v, rows_v,
                     c0, c1, sem_g, sr0, sr1, sw0, sw1):
  wid = _wid()
  base = wid * BPW
  crow = wid * CPW
  pltpu.sync_copy(idx3.at[wid], idx_v)
  gathers = [
      pltpu.async_copy(table.at[idx_v.at[c]],
                       rows_v.at[pl.ds(c * CHUNK, CHUNK)], sem_g)
      for c in range(NCHUNK)
  ]

  bufs = (c0, c1)
  srs = (sr0, sr1)
  sws = (sw0, sw1)

  def rd(i):
    return pltpu.async_copy(table.at[pl.ds(crow + i * CT, CT)],
                            bufs[i % 2], srs[i % 2])

  def wr(i):
    return pltpu.async_copy(bufs[i % 2],
                            mem_out.at[pl.ds(crow + i * CT, CT)], sws[i % 2])

  reads = [None] * CK
  writes = [None] * CK
  reads[0] = rd(0)
  reads[1] = rd(1)
  for i in range(CK):
    reads[i].wait()
    writes[i] = wr(i)
    if i + 2 < CK:
      writes[i].wait()
      reads[i + 2] = rd(i + 2)
  writes[CK - 2].wait()
  writes[CK - 1].wait()

  tail = crow + CK * CT

  @pl.when(wid < NW - 1)
  def _():
    pltpu.sync_copy(table.at[pl.ds(tail, CT)], c0)
    pltpu.sync_copy(c0, mem_out.at[pl.ds(tail, CT)])

  @pl.when(wid == NW - 1)
  def _():
    pltpu.sync_copy(table.at[pl.ds(tail, CT_LAST)], c0.at[pl.ds(0, CT_LAST)])
    pltpu.sync_copy(c0.at[pl.ds(0, CT_LAST)], mem_out.at[pl.ds(tail, CT_LAST)])

  for cp in gathers:
    cp.wait()
  pltpu.sync_copy(rows_v, h_out.at[pl.ds(base, BPW)])


_BLK = 2048


def _gru_body(msg_ref, h_ref, wih_ref, whh_ref, bih_ref, bhh_ref, out_ref):
  x = msg_ref[...]
  h = h_ref[...]
  dn = (((1,), (1,)), ((), ()))
  gi = lax.dot_general(x, wih_ref[...], dn,
                       preferred_element_type=jnp.float32) + bih_ref[...]
  gh = lax.dot_general(h, whh_ref[...], dn,
                       preferred_element_type=jnp.float32) + bhh_ref[...]
  i_r, i_z, i_n = gi[:, :128], gi[:, 128:256], gi[:, 256:]
  h_r, h_z, h_n = gh[:, :128], gh[:, 128:256], gh[:, 256:]
  r = jax.nn.sigmoid(i_r + h_r)
  z = jax.nn.sigmoid(i_z + h_z)
  n = jnp.tanh(i_n + r * h_n)
  out_ref[...] = (1.0 - z) * n + z * h


def _tc_gru(msg, h, w_ih, w_hh, b_ih, b_hh):
  return pl.pallas_call(
      _gru_body,
      grid=(B // _BLK,),
      in_specs=[
          pl.BlockSpec((_BLK, MSG_DIM), lambda i: (i, 0)),
          pl.BlockSpec((_BLK, MEM_DIM), lambda i: (i, 0)),
          pl.BlockSpec((3 * MEM_DIM, MSG_DIM), lambda i: (0, 0)),
          pl.BlockSpec((3 * MEM_DIM, MEM_DIM), lambda i: (0, 0)),
          pl.BlockSpec((3 * MEM_DIM,), lambda i: (0,)),
          pl.BlockSpec((3 * MEM_DIM,), lambda i: (0,)),
      ],
      out_specs=pl.BlockSpec((_BLK, MEM_DIM), lambda i: (i, 0)),
      out_shape=jax.ShapeDtypeStruct((B, MEM_DIM), jnp.float32),
  )(msg, h, w_ih, w_hh, b_ih, b_hh)


@functools.partial(
    pl.kernel,
    mesh=_sc_mesh,
    out_type=jax.ShapeDtypeStruct((N_NODES,), jnp.float32),
    scratch_types=[
        pltpu.VMEM((NCHUNK, CHUNK), jnp.int32),
        pltpu.VMEM((BPW, MEM_DIM), jnp.float32),
        pltpu.VMEM((NW, NCHUNK, CHUNK), jnp.int32),
        pltpu.VMEM((LUW,), jnp.float32),
        pltpu.VMEM((16,), jnp.float32),
        pltpu.SemaphoreType.DMA,
    ],
    compiler_params=_sc_params,
)
def _sc_scatter_lu(idx3, rows, lu, tvec_hbm, mem_ref, lu_out,
                   idxc_v, rows_v, idx_v, seg_v, tv_v, sem):
  wid = _wid()
  base = wid * BPW
  lo = wid * LUW
  last = wid == NW - 1
  pltpu.sync_copy(idx3.at[wid], idxc_v)
  pltpu.sync_copy(rows.at[pl.ds(base, BPW)], rows_v)
  scatters = [
      pltpu.async_copy(rows_v.at[pl.ds(c * CHUNK, CHUNK)],
                       mem_ref.at[idxc_v.at[c]], sem)
      for c in range(NCHUNK)
  ]

  pltpu.sync_copy(idx3, idx_v)

  @pl.when(jnp.logical_not(last))
  def _():
    pltpu.sync_copy(lu.at[pl.ds(lo, LUW)], seg_v)

  @pl.when(last)
  def _():
    pltpu.sync_copy(lu.at[pl.ds(lo, LUW_LAST)], seg_v.at[pl.ds(0, LUW_LAST)])

  pltpu.sync_copy(tvec_hbm, tv_v)
  tv = tv_v[...]
  lov = jnp.full((16,), lo, jnp.int32)
  hiv = lov + jnp.where(last, LUW_LAST, LUW).astype(jnp.int32)

  @pl.loop(0, B // 16, unroll=8)
  def _(i):
    w = i // (NCHUNK * CHUNK // 16)
    rem = i % (NCHUNK * CHUNK // 16)
    c = rem // (CHUNK // 16)
    j = rem % (CHUNK // 16)
    iv = idx_v[w, c, pl.ds(j * 16, 16)]
    m = (iv >= lov) & (iv < hiv)
    plsc.store_scatter(seg_v, [iv - lov], tv, mask=m)

  @pl.when(jnp.logical_not(last))
  def _():
    pltpu.sync_copy(seg_v, lu_out.at[pl.ds(lo, LUW)])

  @pl.when(last)
  def _():
    pltpu.sync_copy(seg_v.at[pl.ds(0, LUW_LAST)], lu_out.at[pl.ds(lo, LUW_LAST)])

  for cp in scatters:
    cp.wait()


def kernel(unique_nids, unique_msg, time, memory, last_update,
           W_ih, W_hh, b_ih, b_hh):
  idx = unique_nids.astype(jnp.int32)
  idx3 = idx.reshape(NW, NCHUNK, CHUNK)
  tvec = jnp.full((16,), time, dtype=jnp.float32)

  h, mem_copy = _sc_gather_clone(memory, idx3)
  h_new = _tc_gru(unique_msg, h, W_ih, W_hh, b_ih, b_hh)

  mem_ref = jax.new_ref(mem_copy)
  updated_last_update = _sc_scatter_lu(idx3, h_new, last_update, tvec, mem_ref)
  updated_memory = jax.freeze(mem_ref)
  return (updated_memory, updated_last_update)


# speedup vs baseline: 1.1464x; 1.0792x over previous
import functools

import jax
import jax.numpy as jnp
from jax import lax
from jax.experimental import pallas as pl
from jax.experimental.pallas import tpu as pltpu
import jax.experimental.pallas.tpu_sc as plsc

N_NODES = 100000
MEM_DIM = 128
MSG_DIM = 256
B = 16384

NC = 2
NS = 16
NW = NC * NS
BPW = B // NW
CHUNK = 128
NCHUNK = BPW // CHUNK

CPW = 3128
CT = 136
CK = 22
CT_LAST = 40

LUW = 3200
LUW_LAST = N_NODES - (NW - 1) * LUW

_sc_mesh = plsc.VectorSubcoreMesh(core_axis_name="c", subcore_axis_name="s")
_sc_params = pltpu.CompilerParams(needs_layout_passes=False)


def _wid():
  return lax.axis_index("s") * NC + lax.axis_index("c")


@functools.partial(
    pl.kernel,
    mesh=_sc_mesh,
    out_type=(
        jax.ShapeDtypeStruct((B, MEM_DIM), jnp.float32),
        jax.ShapeDtypeStruct((N_NODES, MEM_DIM), jnp.float32),
        jax.ShapeDtypeStruct((N_NODES,), jnp.float32),
    ),
    scratch_types=[
        pltpu.VMEM((NCHUNK, CHUNK), jnp.int32),
        pltpu.VMEM((BPW, MEM_DIM), jnp.float32),
        pltpu.VMEM((CT, MEM_DIM), jnp.float32),
        pltpu.VMEM((CT, MEM_DIM), jnp.float32),
        pltpu.VMEM((NW, NCHUNK, CHUNK), jnp.int32),
        pltpu.VMEM((LUW,), jnp.float32),
        pltpu.VMEM((16,), jnp.float32),
        pltpu.SemaphoreType.DMA,
        pltpu.SemaphoreType.DMA,
        pltpu.SemaphoreType.DMA,
        pltpu.SemaphoreType.DMA,
        pltpu.SemaphoreType.DMA,
    ],
    compiler_params=_sc_params,
)
def _sc_gather_clone(table, idx3, lu, tvec_hbm, h_out, mem_out, lu_out,
                     idx_v, rows_v, c0, c1, idx_all, seg_v, tv_v,
                     sem_g, sr0, sr1, sw0, sw1):
  wid = _wid()
  base = wid * BPW
  crow = wid * CPW
  lo = wid * LUW
  last = wid == NW - 1
  pltpu.sync_copy(idx3.at[wid], idx_v)
  gathers = [
      pltpu.async_copy(table.at[idx_v.at[c]],
                       rows_v.at[pl.ds(c * CHUNK, CHUNK)], sem_g)
      for c in range(NCHUNK)
  ]

  bufs = (c0, c1)
  srs = (sr0, sr1)
  sws = (sw0, sw1)

  def rd(i):
    return pltpu.async_copy(table.at[pl.ds(crow + i * CT, CT)],
                            bufs[i % 2], srs[i % 2])

  def wr(i):
    return pltpu.async_copy(bufs[i % 2],
                            mem_out.at[pl.ds(crow + i * CT, CT)], sws[i % 2])

  reads = [None] * CK
  writes = [None] * CK
  reads[0] = rd(0)
  reads[1] = rd(1)

  pltpu.sync_copy(idx3, idx_all)

  @pl.when(jnp.logical_not(last))
  def _():
    pltpu.sync_copy(lu.at[pl.ds(lo, LUW)], seg_v)

  @pl.when(last)
  def _():
    pltpu.sync_copy(lu.at[pl.ds(lo, LUW_LAST)], seg_v.at[pl.ds(0, LUW_LAST)])

  pltpu.sync_copy(tvec_hbm, tv_v)
  tv = tv_v[...]
  lov = jnp.full((16,), lo, jnp.int32)
  hiv = lov + jnp.where(last, LUW_LAST, LUW).astype(jnp.int32)

  nvreg = B // 16
  per_it = nvreg // CK

  def scan(vlo, vhi):
    @pl.loop(vlo, vhi)
    def _(v):
      w = v // (NCHUNK * CHUNK // 16)
      rem = v % (NCHUNK * CHUNK // 16)
      c = rem // (CHUNK // 16)
      j = rem % (CHUNK // 16)
      iv = idx_all[w, c, pl.ds(j * 16, 16)]
      m = (iv >= lov) & (iv < hiv)
      plsc.store_scatter(seg_v, [iv - lov], tv, mask=m)

  for i in range(CK):
    reads[i].wait()
    writes[i] = wr(i)
    scan(i * per_it, (i + 1) * per_it)
    if i + 2 < CK:
      writes[i].wait()
      reads[i + 2] = rd(i + 2)
  writes[CK - 2].wait()
  writes[CK - 1].wait()
  scan(CK * per_it, nvreg)

  @pl.when(jnp.logical_not(last))
  def _():
    pltpu.sync_copy(seg_v, lu_out.at[pl.ds(lo, LUW)])

  @pl.when(last)
  def _():
    pltpu.sync_copy(seg_v.at[pl.ds(0, LUW_LAST)], lu_out.at[pl.ds(lo, LUW_LAST)])

  tail = crow + CK * CT

  @pl.when(wid < NW - 1)
  def _():
    pltpu.sync_copy(table.at[pl.ds(tail, CT)], c0)
    pltpu.sync_copy(c0, mem_out.at[pl.ds(tail, CT)])

  @pl.when(wid == NW - 1)
  def _():
    pltpu.sync_copy(table.at[pl.ds(tail, CT_LAST)], c0.at[pl.ds(0, CT_LAST)])
    pltpu.sync_copy(c0.at[pl.ds(0, CT_LAST)], mem_out.at[pl.ds(tail, CT_LAST)])

  for cp in gathers:
    cp.wait()
  pltpu.sync_copy(rows_v, h_out.at[pl.ds(base, BPW)])


_BLK = 2048


def _gru_body(msg_ref, h_ref, wih_ref, whh_ref, bih_ref, bhh_ref, out_ref):
  x = msg_ref[...]
  h = h_ref[...]
  dn = (((1,), (1,)), ((), ()))
  gi = lax.dot_general(x, wih_ref[...], dn,
                       preferred_element_type=jnp.float32) + bih_ref[...]
  gh = lax.dot_general(h, whh_ref[...], dn,
                       preferred_element_type=jnp.float32) + bhh_ref[...]
  i_r, i_z, i_n = gi[:, :128], gi[:, 128:256], gi[:, 256:]
  h_r, h_z, h_n = gh[:, :128], gh[:, 128:256], gh[:, 256:]
  r = jax.nn.sigmoid(i_r + h_r)
  z = jax.nn.sigmoid(i_z + h_z)
  n = jnp.tanh(i_n + r * h_n)
  out_ref[...] = (1.0 - z) * n + z * h


def _tc_gru(msg, h, w_ih, w_hh, b_ih, b_hh):
  return pl.pallas_call(
      _gru_body,
      grid=(B // _BLK,),
      in_specs=[
          pl.BlockSpec((_BLK, MSG_DIM), lambda i: (i, 0)),
          pl.BlockSpec((_BLK, MEM_DIM), lambda i: (i, 0)),
          pl.BlockSpec((3 * MEM_DIM, MSG_DIM), lambda i: (0, 0)),
          pl.BlockSpec((3 * MEM_DIM, MEM_DIM), lambda i: (0, 0)),
          pl.BlockSpec((3 * MEM_DIM,), lambda i: (0,)),
          pl.BlockSpec((3 * MEM_DIM,), lambda i: (0,)),
      ],
      out_specs=pl.BlockSpec((_BLK, MEM_DIM), lambda i: (i, 0)),
      out_shape=jax.ShapeDtypeStruct((B, MEM_DIM), jnp.float32),
  )(msg, h, w_ih, w_hh, b_ih, b_hh)


@functools.partial(
    pl.kernel,
    mesh=_sc_mesh,
    out_type=(),
    scratch_types=[
        pltpu.VMEM((NCHUNK, CHUNK), jnp.int32),
        pltpu.VMEM((BPW, MEM_DIM), jnp.float32),
        pltpu.SemaphoreType.DMA,
    ],
)
def _sc_scatter(idx3, rows, mem_ref, idxc_v, rows_v, sem):
  wid = _wid()
  base = wid * BPW
  pltpu.sync_copy(idx3.at[wid], idxc_v)
  pltpu.sync_copy(rows.at[pl.ds(base, BPW)], rows_v)
  scatters = [
      pltpu.async_copy(rows_v.at[pl.ds(c * CHUNK, CHUNK)],
                       mem_ref.at[idxc_v.at[c]], sem)
      for c in range(NCHUNK)
  ]
  for cp in scatters:
    cp.wait()


def kernel(unique_nids, unique_msg, time, memory, last_update,
           W_ih, W_hh, b_ih, b_hh):
  idx = unique_nids.astype(jnp.int32)
  idx3 = idx.reshape(NW, NCHUNK, CHUNK)
  tvec = jnp.full((16,), time, dtype=jnp.float32)

  h, mem_copy, updated_last_update = _sc_gather_clone(
      memory, idx3, last_update, tvec)
  h_new = _tc_gru(unique_msg, h, W_ih, W_hh, b_ih, b_hh)

  mem_ref = jax.new_ref(mem_copy)
  _sc_scatter(idx3, h_new, mem_ref)
  updated_memory = jax.freeze(mem_ref)
  return (updated_memory, updated_last_update)


# speedup vs baseline: 1.1605x; 1.0123x over previous
import functools

import jax
import jax.numpy as jnp
from jax import lax
from jax.experimental import pallas as pl
from jax.experimental.pallas import tpu as pltpu
import jax.experimental.pallas.tpu_sc as plsc

N_NODES = 100000
MEM_DIM = 128
MSG_DIM = 256
B = 16384

NC = 2
NS = 16
NW = NC * NS
BPW = B // NW
CHUNK = 128
NCHUNK = BPW // CHUNK

CPW = 3128
CT = 136
CK = 22
CT_LAST = 40

LUW = 3200
LUW_LAST = N_NODES - (NW - 1) * LUW

_sc_mesh = plsc.VectorSubcoreMesh(core_axis_name="c", subcore_axis_name="s")
_sc_params = pltpu.CompilerParams(needs_layout_passes=False)


def _wid():
  return lax.axis_index("s") * NC + lax.axis_index("c")


@functools.partial(
    pl.kernel,
    mesh=_sc_mesh,
    out_type=(
        jax.ShapeDtypeStruct((B, MEM_DIM), jnp.float32),
        jax.ShapeDtypeStruct((N_NODES, MEM_DIM), jnp.float32),
        jax.ShapeDtypeStruct((N_NODES,), jnp.float32),
    ),
    scratch_types=[
        pltpu.VMEM((NCHUNK, CHUNK), jnp.int32),
        pltpu.VMEM((BPW, MEM_DIM), jnp.float32),
        pltpu.VMEM((CT, MEM_DIM), jnp.float32),
        pltpu.VMEM((CT, MEM_DIM), jnp.float32),
        pltpu.VMEM((NW, NCHUNK, CHUNK), jnp.int32),
        pltpu.VMEM((LUW,), jnp.float32),
        pltpu.VMEM((16,), jnp.float32),
        pltpu.SemaphoreType.DMA,
        pltpu.SemaphoreType.DMA,
        pltpu.SemaphoreType.DMA,
        pltpu.SemaphoreType.DMA,
        pltpu.SemaphoreType.DMA,
    ],
    compiler_params=_sc_params,
)
def _sc_gather_clone(table, idx3, lu, tvec_hbm, h_out, mem_out, lu_out,
                     idx_v, rows_v, c0, c1, idx_all, seg_v, tv_v,
                     sem_g, sr0, sr1, sw0, sw1):
  wid = _wid()
  base = wid * BPW
  crow = wid * CPW
  lo = wid * LUW
  last = wid == NW - 1
  pltpu.sync_copy(idx3.at[wid], idx_v)
  gathers = [
      pltpu.async_copy(table.at[idx_v.at[c]],
                       rows_v.at[pl.ds(c * CHUNK, CHUNK)], sem_g)
      for c in range(NCHUNK)
  ]

  bufs = (c0, c1)
  srs = (sr0, sr1)
  sws = (sw0, sw1)

  def rd(i):
    return pltpu.async_copy(table.at[pl.ds(crow + i * CT, CT)],
                            bufs[i % 2], srs[i % 2])

  def wr(i):
    return pltpu.async_copy(bufs[i % 2],
                            mem_out.at[pl.ds(crow + i * CT, CT)], sws[i % 2])

  reads = [None] * CK
  writes = [None] * CK
  reads[0] = rd(0)
  reads[1] = rd(1)

  pltpu.sync_copy(idx3, idx_all)

  @pl.when(jnp.logical_not(last))
  def _():
    pltpu.sync_copy(lu.at[pl.ds(lo, LUW)], seg_v)

  @pl.when(last)
  def _():
    pltpu.sync_copy(lu.at[pl.ds(lo, LUW_LAST)], seg_v.at[pl.ds(0, LUW_LAST)])

  pltpu.sync_copy(tvec_hbm, tv_v)
  tv = tv_v[...]
  lov = jnp.full((16,), lo, jnp.int32)
  hiv = lov + jnp.where(last, LUW_LAST, LUW).astype(jnp.int32)

  nvreg = B // 16
  per_it = nvreg // CK

  def scan(vlo, vhi):
    @pl.loop(vlo, vhi)
    def _(v):
      w = v // (NCHUNK * CHUNK // 16)
      rem = v % (NCHUNK * CHUNK // 16)
      c = rem // (CHUNK // 16)
      j = rem % (CHUNK // 16)
      iv = idx_all[w, c, pl.ds(j * 16, 16)]
      m = (iv >= lov) & (iv < hiv)
      plsc.store_scatter(seg_v, [iv - lov], tv, mask=m)

  for i in range(CK):
    reads[i].wait()
    writes[i] = wr(i)
    scan(i * per_it, (i + 1) * per_it)
    if i + 2 < CK:
      writes[i].wait()
      reads[i + 2] = rd(i + 2)
  writes[CK - 2].wait()
  writes[CK - 1].wait()
  scan(CK * per_it, nvreg)

  @pl.when(jnp.logical_not(last))
  def _():
    pltpu.sync_copy(seg_v, lu_out.at[pl.ds(lo, LUW)])

  @pl.when(last)
  def _():
    pltpu.sync_copy(seg_v.at[pl.ds(0, LUW_LAST)], lu_out.at[pl.ds(lo, LUW_LAST)])

  tail = crow + CK * CT

  @pl.when(wid < NW - 1)
  def _():
    pltpu.sync_copy(table.at[pl.ds(tail, CT)], c0)
    pltpu.sync_copy(c0, mem_out.at[pl.ds(tail, CT)])

  @pl.when(wid == NW - 1)
  def _():
    pltpu.sync_copy(table.at[pl.ds(tail, CT_LAST)], c0.at[pl.ds(0, CT_LAST)])
    pltpu.sync_copy(c0.at[pl.ds(0, CT_LAST)], mem_out.at[pl.ds(tail, CT_LAST)])

  for cp in gathers:
    cp.wait()
  pltpu.sync_copy(rows_v, h_out.at[pl.ds(base, BPW)])


_BLK = 4096


def _gru_body(msg_ref, h_ref, wih_ref, whh_ref, bih_ref, bhh_ref, out_ref):
  x = msg_ref[...]
  h = h_ref[...]
  dn = (((1,), (1,)), ((), ()))
  gi = lax.dot_general(x, wih_ref[...], dn,
                       preferred_element_type=jnp.float32) + bih_ref[...]
  gh = lax.dot_general(h, whh_ref[...], dn,
                       preferred_element_type=jnp.float32) + bhh_ref[...]
  i_r, i_z, i_n = gi[:, :128], gi[:, 128:256], gi[:, 256:]
  h_r, h_z, h_n = gh[:, :128], gh[:, 128:256], gh[:, 256:]
  r = jax.nn.sigmoid(i_r + h_r)
  z = jax.nn.sigmoid(i_z + h_z)
  n = jnp.tanh(i_n + r * h_n)
  out_ref[...] = (1.0 - z) * n + z * h


def _tc_gru(msg, h, w_ih, w_hh, b_ih, b_hh):
  return pl.pallas_call(
      _gru_body,
      grid=(B // _BLK,),
      in_specs=[
          pl.BlockSpec((_BLK, MSG_DIM), lambda i: (i, 0)),
          pl.BlockSpec((_BLK, MEM_DIM), lambda i: (i, 0)),
          pl.BlockSpec((3 * MEM_DIM, MSG_DIM), lambda i: (0, 0)),
          pl.BlockSpec((3 * MEM_DIM, MEM_DIM), lambda i: (0, 0)),
          pl.BlockSpec((3 * MEM_DIM,), lambda i: (0,)),
          pl.BlockSpec((3 * MEM_DIM,), lambda i: (0,)),
      ],
      out_specs=pl.BlockSpec((_BLK, MEM_DIM), lambda i: (i, 0)),
      out_shape=jax.ShapeDtypeStruct((B, MEM_DIM), jnp.float32),
  )(msg, h, w_ih, w_hh, b_ih, b_hh)


@functools.partial(
    pl.kernel,
    mesh=_sc_mesh,
    out_type=(),
    scratch_types=[
        pltpu.VMEM((NCHUNK, CHUNK), jnp.int32),
        pltpu.VMEM((BPW, MEM_DIM), jnp.float32),
        pltpu.SemaphoreType.DMA,
    ],
)
def _sc_scatter(idx3, rows, mem_ref, idxc_v, rows_v, sem):
  wid = _wid()
  base = wid * BPW
  pltpu.sync_copy(idx3.at[wid], idxc_v)
  pltpu.sync_copy(rows.at[pl.ds(base, BPW)], rows_v)
  scatters = [
      pltpu.async_copy(rows_v.at[pl.ds(c * CHUNK, CHUNK)],
                       mem_ref.at[idxc_v.at[c]], sem)
      for c in range(NCHUNK)
  ]
  for cp in scatters:
    cp.wait()


def kernel(unique_nids, unique_msg, time, memory, last_update,
           W_ih, W_hh, b_ih, b_hh):
  idx = unique_nids.astype(jnp.int32)
  idx3 = idx.reshape(NW, NCHUNK, CHUNK)
  tvec = jnp.full((16,), time, dtype=jnp.float32)

  h, mem_copy, updated_last_update = _sc_gather_clone(
      memory, idx3, last_update, tvec)
  h_new = _tc_gru(unique_msg, h, W_ih, W_hh, b_ih, b_hh)

  mem_ref = jax.new_ref(mem_copy)
  _sc_scatter(idx3, h_new, mem_ref)
  updated_memory = jax.freeze(mem_ref)
  return (updated_memory, updated_last_update)
